# baseline (speedup 1.0000x reference)
import functools

import jax
import jax.numpy as jnp
from jax import lax
from jax.experimental import pallas as pl
from jax.experimental.pallas import tpu as pltpu
from jax.experimental.pallas import tpu_sc as plsc

N = 10000
E = 320000
D = 128
EPS = 0.1
GAMMA = 0.1

NC = 2
NS = 16
NW = NC * NS
EPT = E // NW
K = 112
CH = 90
EPP = CH * K
NACC = N + 16
STRIPE = 624
TAIL = N - NS * STRIPE
TAIL0 = NS * STRIPE
ZR = 208
NH = 8

_F32 = jnp.float32
_I32 = jnp.int32


def _sc_mesh():
    return plsc.VectorSubcoreMesh(core_axis_name="c", subcore_axis_name="s")


def _zero_stripe(zeros_hbm, acc_sh, s):
    row0 = s * STRIPE
    for t in range(STRIPE // ZR):
        pltpu.sync_copy(zeros_hbm, acc_sh.at[pl.ds(row0 + t * ZR, ZR)])

    @pl.when(s == 0)
    def _():
        pltpu.sync_copy(zeros_hbm.at[pl.ds(0, TAIL)], acc_sh.at[pl.ds(TAIL0, TAIL)])


def _write_stripe(acc_sh, out_hbm, c, s):
    row0 = s * STRIPE
    pltpu.sync_copy(acc_sh.at[pl.ds(row0, STRIPE)],
                    out_hbm.at[c, pl.ds(row0, STRIPE)])

    @pl.when(s == 0)
    def _():
        pltpu.sync_copy(acc_sh.at[pl.ds(TAIL0, TAIL)],
                        out_hbm.at[c, pl.ds(TAIL0, TAIL)])


def _make_deg_kernel():
    @functools.partial(
        pl.kernel,
        mesh=_sc_mesh(),
        out_type=jax.ShapeDtypeStruct((NC, N, D), _F32),
        scratch_types=[
            pltpu.VMEM((CH, K), _I32),
            pltpu.VMEM((K, D), _F32),
            pltpu.VMEM_SHARED((NACC, D), _F32),
        ],
    )
    def deg_kernel(pk_hbm, ones_hbm, zeros_hbm, degp_hbm, dst_v, ones_v, acc_sh):
        c = lax.axis_index("c")
        s = lax.axis_index("s")
        w = s * NC + c
        _zero_stripe(zeros_hbm, acc_sh, s)
        pltpu.sync_copy(pk_hbm.at[w], dst_v)
        pltpu.sync_copy(ones_hbm, ones_v)

        def unpack(j, carry):
            for k in range(K // 16):
                sl = pl.ds(k * 16, 16)
                dst_v[j, sl] = dst_v[j, sl] >> 16
            return carry

        lax.fori_loop(0, CH, unpack, 0)
        plsc.subcore_barrier()

        def body(j, carry):
            pltpu.sync_copy(ones_v, acc_sh.at[dst_v.at[j]], add=True)
            return carry

        lax.fori_loop(0, CH, body, 0)
        plsc.subcore_barrier()
        _write_stripe(acc_sh, degp_hbm, c, s)

    return deg_kernel


def _make_agg_kernel():
    @functools.partial(
        pl.kernel,
        mesh=_sc_mesh(),
        out_type=jax.ShapeDtypeStruct((NC, N, D), _F32),
        scratch_types=[
            pltpu.VMEM((CH, K), _I32),
            pltpu.VMEM((2, K), _I32),
            pltpu.VMEM((2, K), _I32),
            pltpu.VMEM((2, K, D), _F32),
            pltpu.VMEM_SHARED((NACC, D), _F32),
            pltpu.SemaphoreType.DMA,
        ],
    )
    def agg_kernel(hs_hbm, pk_hbm, zeros_hbm, part_hbm, pk_v, src_c, dst_c,
                   rows_v, acc_sh, sem):
        c = lax.axis_index("c")
        s = lax.axis_index("s")
        w = s * NC + c
        _zero_stripe(zeros_hbm, acc_sh, s)
        pltpu.sync_copy(pk_hbm.at[w], pk_v)

        def unpack(j, b):
            for k in range(K // 16):
                sl = pl.ds(k * 16, 16)
                v = pk_v[j, sl]
                src_c[b, sl] = v & 0xFFFF
                dst_c[b, sl] = v >> 16

        plsc.subcore_barrier()

        unpack(0, 0)
        pltpu.make_async_copy(hs_hbm.at[src_c.at[0]], rows_v.at[0], sem).start()

        def pair(jo, carry):
            j0 = 2 * jo
            unpack(j0 + 1, 1)
            pltpu.make_async_copy(hs_hbm.at[src_c.at[0]], rows_v.at[0], sem).wait()
            pltpu.make_async_copy(hs_hbm.at[src_c.at[1]], rows_v.at[1], sem).start()
            pltpu.sync_copy(rows_v.at[0], acc_sh.at[dst_c.at[0]], add=True)
            pltpu.make_async_copy(hs_hbm.at[src_c.at[1]], rows_v.at[1], sem).wait()

            @pl.when(j0 + 2 < CH)
            def _():
                unpack(j0 + 2, 0)
                pltpu.make_async_copy(hs_hbm.at[src_c.at[0]], rows_v.at[0], sem).start()

            pltpu.sync_copy(rows_v.at[1], acc_sh.at[dst_c.at[1]], add=True)
            return carry

        lax.fori_loop(0, CH // 2, pair, 0)
        if CH % 2:
            pltpu.make_async_copy(hs_hbm.at[src_c.at[0]], rows_v.at[0], sem).wait()
            pltpu.sync_copy(rows_v.at[0], acc_sh.at[dst_c.at[0]], add=True)
        plsc.subcore_barrier()
        _write_stripe(acc_sh, part_hbm, c, s)

    return agg_kernel


_deg_kernel = _make_deg_kernel()
_agg_kernel = _make_agg_kernel()

BN = 1000


def _tc1_body(x_ref, lin_ref, degp_ref, hs_ref, dinv_ref):
    deg = 1.0 + degp_ref[0] + degp_ref[1]
    dinv = lax.rsqrt(deg)
    h = jnp.dot(x_ref[...], lin_ref[...].T, preferred_element_type=_F32)
    hs_ref[...] = h * dinv
    dinv_ref[...] = dinv[:, 0:16]


def _tc2_body(x_ref, hs_ref, part_ref, dinv_ref, W_ref, b_ref, lin2_ref,
              x1_ref, hs2_ref):
    xx = x_ref[...]
    dinv = dinv_ref[:, 0:1]
    agg = part_ref[0] + part_ref[1] + hs_ref[...]
    W = W_ref[...]
    z = (jnp.dot(xx, W.T, preferred_element_type=_F32)
         - jnp.dot(xx, W, preferred_element_type=_F32)
         - GAMMA * xx + dinv * agg + b_ref[...])
    x1 = jax.nn.relu(xx + EPS * jnp.tanh(z))
    x1_ref[...] = x1
    hs2_ref[...] = jnp.dot(x1, lin2_ref[...].T,
                           preferred_element_type=_F32) * dinv


def _tc3_body(x_ref, hs_ref, part_ref, dinv_ref, W_ref, b_ref, ow_ref, ob_ref,
              out_ref):
    xx = x_ref[...]
    dinv = dinv_ref[:, 0:1]
    agg = part_ref[0] + part_ref[1] + hs_ref[...]
    W = W_ref[...]
    z = (jnp.dot(xx, W.T, preferred_element_type=_F32)
         - jnp.dot(xx, W, preferred_element_type=_F32)
         - GAMMA * xx + dinv * agg + b_ref[...])
    x2 = jax.nn.relu(xx + EPS * jnp.tanh(z))
    out_ref[...] = jnp.dot(x2, ow_ref[...], preferred_element_type=_F32) + ob_ref[0, 0]


def _row_spec(width):
    return pl.BlockSpec((BN, width), lambda i: (i, 0))


def _full_spec(shape):
    nd = len(shape)
    return pl.BlockSpec(shape, lambda i: (0,) * nd)


def _part_spec(width):
    return pl.BlockSpec((NC, BN, width), lambda i: (0, i, 0))


def kernel(x, edge_index, W1, b1, lin1, W2, b2, lin2, out_w, out_b):
    src = edge_index[0].astype(_I32)
    dst = edge_index[1].astype(_I32)
    pk = (src | (dst << 16)).reshape(NW, EPT)
    pk = jnp.pad(pk, ((0, 0), (0, EPP - EPT)),
                 constant_values=N << 16).reshape(NW, CH, K)
    onesD = jnp.ones((K, D), _F32)
    zerosD = jnp.zeros((ZR, D), _F32)
    b1r = b1.reshape(1, D)
    b2r = b2.reshape(1, D)
    obr = out_b.reshape(1, 1)
    grid = (N // BN,)

    degp = _deg_kernel(pk, onesD, zerosD)

    hs1, dinv16 = pl.pallas_call(
        _tc1_body,
        grid=grid,
        in_specs=[_row_spec(D), _full_spec((D, D)), _part_spec(D)],
        out_specs=[_row_spec(D), _row_spec(16)],
        out_shape=[jax.ShapeDtypeStruct((N, D), _F32),
                   jax.ShapeDtypeStruct((N, 16), _F32)],
    )(x, lin1, degp)

    part1 = _agg_kernel(hs1, pk, zerosD)

    x1, hs2 = pl.pallas_call(
        _tc2_body,
        grid=grid,
        in_specs=[_row_spec(D), _row_spec(D), _part_spec(D), _row_spec(16),
                  _full_spec((D, D)), _full_spec((1, D)), _full_spec((D, D))],
        out_specs=[_row_spec(D), _row_spec(D)],
        out_shape=[jax.ShapeDtypeStruct((N, D), _F32),
                   jax.ShapeDtypeStruct((N, D), _F32)],
    )(x, hs1, part1, dinv16, W1, b1r, lin2)

    part2 = _agg_kernel(hs2, pk, zerosD)

    out2d = pl.pallas_call(
        _tc3_body,
        grid=grid,
        in_specs=[_row_spec(D), _row_spec(D), _part_spec(D), _row_spec(16),
                  _full_spec((D, D)), _full_spec((1, D)), _full_spec((D, 1)),
                  pl.BlockSpec(memory_space=pltpu.SMEM)],
        out_specs=pl.BlockSpec((BN, 1), lambda i: (i, 0)),
        out_shape=jax.ShapeDtypeStruct((N, 1), _F32),
    )(x1, hs2, part2, dinv16, W2, b2r, out_w.reshape(D, 1), obr)

    return out2d[:, 0]


# speedup vs baseline: 1.0016x; 1.0016x over previous
import functools

import jax
import jax.numpy as jnp
from jax import lax
from jax.experimental import pallas as pl
from jax.experimental.pallas import tpu as pltpu
from jax.experimental.pallas import tpu_sc as plsc

N = 10000
E = 320000
D = 128
EPS = 0.1
GAMMA = 0.1

NC = 2
NS = 16
NW = NC * NS
EPT = E // NW
K = 112
CH = 90
EPP = CH * K
NACC = N + 16
STRIPE = 624
TAIL = N - NS * STRIPE
TAIL0 = NS * STRIPE
ZR = 208
NH = 8

_F32 = jnp.float32
_I32 = jnp.int32


def _sc_mesh():
    return plsc.VectorSubcoreMesh(core_axis_name="c", subcore_axis_name="s")


def _zero_stripe(zeros_hbm, acc_sh, s):
    row0 = s * STRIPE
    for t in range(STRIPE // ZR):
        pltpu.sync_copy(zeros_hbm, acc_sh.at[pl.ds(row0 + t * ZR, ZR)])

    @pl.when(s == 0)
    def _():
        pltpu.sync_copy(zeros_hbm.at[pl.ds(0, TAIL)], acc_sh.at[pl.ds(TAIL0, TAIL)])


def _write_stripe(acc_sh, out_hbm, c, s):
    row0 = s * STRIPE
    pltpu.sync_copy(acc_sh.at[pl.ds(row0, STRIPE)],
                    out_hbm.at[c, pl.ds(row0, STRIPE)])

    @pl.when(s == 0)
    def _():
        pltpu.sync_copy(acc_sh.at[pl.ds(TAIL0, TAIL)],
                        out_hbm.at[c, pl.ds(TAIL0, TAIL)])


def _make_deg_kernel():
    @functools.partial(
        pl.kernel,
        mesh=_sc_mesh(),
        out_type=jax.ShapeDtypeStruct((NC, N, D), _F32),
        scratch_types=[
            pltpu.VMEM((CH, K), _I32),
            pltpu.VMEM((K, D), _F32),
            pltpu.VMEM_SHARED((NACC, D), _F32),
        ],
    )
    def deg_kernel(pk_hbm, ones_hbm, zeros_hbm, degp_hbm, dst_v, ones_v, acc_sh):
        c = lax.axis_index("c")
        s = lax.axis_index("s")
        w = s * NC + c
        _zero_stripe(zeros_hbm, acc_sh, s)
        pltpu.sync_copy(pk_hbm.at[w], dst_v)
        pltpu.sync_copy(ones_hbm, ones_v)

        def unpack(j, carry):
            for k in range(K // 16):
                sl = pl.ds(k * 16, 16)
                dst_v[j, sl] = dst_v[j, sl] >> 16
            return carry

        lax.fori_loop(0, CH, unpack, 0)
        plsc.subcore_barrier()

        def body(j, carry):
            pltpu.sync_copy(ones_v, acc_sh.at[dst_v.at[j]], add=True)
            return carry

        lax.fori_loop(0, CH, body, 0)
        plsc.subcore_barrier()
        _write_stripe(acc_sh, degp_hbm, c, s)

    return deg_kernel


def _make_agg_kernel():
    @functools.partial(
        pl.kernel,
        mesh=_sc_mesh(),
        out_type=jax.ShapeDtypeStruct((NC, N, D), _F32),
        scratch_types=[
            pltpu.VMEM((CH, K), _I32),
            pltpu.VMEM((2, K), _I32),
            pltpu.VMEM((2, K), _I32),
            pltpu.VMEM((2, K, D), _F32),
            pltpu.VMEM_SHARED((NACC, D), _F32),
            pltpu.SemaphoreType.DMA,
        ],
    )
    def agg_kernel(hs_hbm, pk_hbm, zeros_hbm, part_hbm, pk_v, src_c, dst_c,
                   rows_v, acc_sh, sem):
        c = lax.axis_index("c")
        s = lax.axis_index("s")
        w = s * NC + c
        _zero_stripe(zeros_hbm, acc_sh, s)
        pltpu.sync_copy(pk_hbm.at[w], pk_v)

        def unpack(j, b):
            for k in range(K // 16):
                sl = pl.ds(k * 16, 16)
                v = pk_v[j, sl]
                src_c[b, sl] = v & 0xFFFF
                dst_c[b, sl] = v >> 16

        plsc.subcore_barrier()

        unpack(0, 0)
        pltpu.make_async_copy(hs_hbm.at[src_c.at[0]], rows_v.at[0], sem).start()

        def pair(jo, carry):
            j0 = 2 * jo
            unpack(j0 + 1, 1)
            pltpu.make_async_copy(hs_hbm.at[src_c.at[0]], rows_v.at[0], sem).wait()
            pltpu.make_async_copy(hs_hbm.at[src_c.at[1]], rows_v.at[1], sem).start()
            pltpu.sync_copy(rows_v.at[0], acc_sh.at[dst_c.at[0]], add=True)
            pltpu.make_async_copy(hs_hbm.at[src_c.at[1]], rows_v.at[1], sem).wait()

            @pl.when(j0 + 2 < CH)
            def _():
                unpack(j0 + 2, 0)
                pltpu.make_async_copy(hs_hbm.at[src_c.at[0]], rows_v.at[0], sem).start()

            pltpu.sync_copy(rows_v.at[1], acc_sh.at[dst_c.at[1]], add=True)
            return carry

        lax.fori_loop(0, CH // 2, pair, 0)
        if CH % 2:
            pltpu.make_async_copy(hs_hbm.at[src_c.at[0]], rows_v.at[0], sem).wait()
            pltpu.sync_copy(rows_v.at[0], acc_sh.at[dst_c.at[0]], add=True)
        plsc.subcore_barrier()
        _write_stripe(acc_sh, part_hbm, c, s)

    return agg_kernel


_deg_kernel = _make_deg_kernel()
_agg_kernel = _make_agg_kernel()

BN = 1000


def _tc1_body(x_ref, lin_ref, degp_ref, hs_ref, dinv_ref):
    deg = 1.0 + degp_ref[0] + degp_ref[1]
    dinv = lax.rsqrt(deg)
    h = jnp.dot(x_ref[...], lin_ref[...].T, preferred_element_type=_F32)
    hs_ref[...] = h * dinv
    dinv_ref[...] = dinv[:, 0:16]


def _tc2_body(x_ref, hs_ref, part_ref, dinv_ref, W_ref, b_ref, lin2_ref,
              x1_ref, hs2_ref):
    xx = x_ref[...]
    dinv = dinv_ref[:, 0:1]
    agg = part_ref[0] + part_ref[1] + hs_ref[...]
    W = W_ref[...]
    z = (jnp.dot(xx, W.T, preferred_element_type=_F32)
         - jnp.dot(xx, W, preferred_element_type=_F32)
         - GAMMA * xx + dinv * agg + b_ref[...])
    x1 = jax.nn.relu(xx + EPS * jnp.tanh(z))
    x1_ref[...] = x1
    hs2_ref[...] = jnp.dot(x1, lin2_ref[...].T,
                           preferred_element_type=_F32) * dinv


def _tc3_body(x_ref, hs_ref, part_ref, dinv_ref, W_ref, b_ref, ow_ref, ob_ref,
              out_ref):
    xx = x_ref[...]
    dinv = dinv_ref[:, 0:1]
    agg = part_ref[0] + part_ref[1] + hs_ref[...]
    W = W_ref[...]
    z = (jnp.dot(xx, W.T, preferred_element_type=_F32)
         - jnp.dot(xx, W, preferred_element_type=_F32)
         - GAMMA * xx + dinv * agg + b_ref[...])
    x2 = jax.nn.relu(xx + EPS * jnp.tanh(z))
    out_ref[...] = jnp.dot(x2, ow_ref[...], preferred_element_type=_F32) + ob_ref[0, 0]


def _row_spec(width):
    return pl.BlockSpec((BN, width), lambda i: (i, 0))


def _full_spec(shape):
    nd = len(shape)
    return pl.BlockSpec(shape, lambda i: (0,) * nd)


def _part_spec(width):
    return pl.BlockSpec((NC, BN, width), lambda i: (0, i, 0))


def kernel(x, edge_index, W1, b1, lin1, W2, b2, lin2, out_w, out_b):
    src = edge_index[0].astype(_I32)
    dst = edge_index[1].astype(_I32)
    pk = (src | (dst << 16)).reshape(NW, EPT)
    junk = (N + jnp.arange(NW, dtype=_I32) // NC) << 16
    pk = jnp.concatenate(
        [pk, jnp.broadcast_to(junk[:, None], (NW, EPP - EPT))],
        axis=1).reshape(NW, CH, K)
    onesD = jnp.ones((K, D), _F32)
    zerosD = jnp.zeros((ZR, D), _F32)
    b1r = b1.reshape(1, D)
    b2r = b2.reshape(1, D)
    obr = out_b.reshape(1, 1)
    grid = (N // BN,)

    degp = _deg_kernel(pk, onesD, zerosD)

    hs1, dinv16 = pl.pallas_call(
        _tc1_body,
        grid=grid,
        in_specs=[_row_spec(D), _full_spec((D, D)), _part_spec(D)],
        out_specs=[_row_spec(D), _row_spec(16)],
        out_shape=[jax.ShapeDtypeStruct((N, D), _F32),
                   jax.ShapeDtypeStruct((N, 16), _F32)],
    )(x, lin1, degp)

    part1 = _agg_kernel(hs1, pk, zerosD)

    x1, hs2 = pl.pallas_call(
        _tc2_body,
        grid=grid,
        in_specs=[_row_spec(D), _row_spec(D), _part_spec(D), _row_spec(16),
                  _full_spec((D, D)), _full_spec((1, D)), _full_spec((D, D))],
        out_specs=[_row_spec(D), _row_spec(D)],
        out_shape=[jax.ShapeDtypeStruct((N, D), _F32),
                   jax.ShapeDtypeStruct((N, D), _F32)],
    )(x, hs1, part1, dinv16, W1, b1r, lin2)

    part2 = _agg_kernel(hs2, pk, zerosD)

    out2d = pl.pallas_call(
        _tc3_body,
        grid=grid,
        in_specs=[_row_spec(D), _row_spec(D), _part_spec(D), _row_spec(16),
                  _full_spec((D, D)), _full_spec((1, D)), _full_spec((D, 1)),
                  pl.BlockSpec(memory_space=pltpu.SMEM)],
        out_specs=pl.BlockSpec((BN, 1), lambda i: (i, 0)),
        out_shape=jax.ShapeDtypeStruct((N, 1), _F32),
    )(x1, hs2, part2, dinv16, W2, b2r, out_w.reshape(D, 1), obr)

    return out2d[:, 0]


# speedup vs baseline: 1.2922x; 1.2901x over previous
import functools

import jax
import jax.numpy as jnp
from jax import lax
from jax.experimental import pallas as pl
from jax.experimental.pallas import tpu as pltpu
from jax.experimental.pallas import tpu_sc as plsc

N = 10000
E = 320000
D = 128
EPS = 0.1
GAMMA = 0.1

NC = 2
NS = 16
NW = NC * NS
EPT = E // NW
K = 80
CH = EPT // K
NACC = N
STRIPE = 624
TAIL = N - NS * STRIPE
TAIL0 = NS * STRIPE
ZR = 208
NH = 8

_F32 = jnp.float32
_I32 = jnp.int32


def _sc_mesh():
    return plsc.VectorSubcoreMesh(core_axis_name="c", subcore_axis_name="s")


def _zero_stripe(zeros_hbm, acc_sh, s):
    row0 = s * STRIPE
    for t in range(STRIPE // ZR):
        pltpu.sync_copy(zeros_hbm, acc_sh.at[pl.ds(row0 + t * ZR, ZR)])

    @pl.when(s == 0)
    def _():
        pltpu.sync_copy(zeros_hbm.at[pl.ds(0, TAIL)], acc_sh.at[pl.ds(TAIL0, TAIL)])


def _write_stripe(acc_sh, out_hbm, c, s):
    row0 = s * STRIPE
    pltpu.sync_copy(acc_sh.at[pl.ds(row0, STRIPE)],
                    out_hbm.at[c, pl.ds(row0, STRIPE)])

    @pl.when(s == 0)
    def _():
        pltpu.sync_copy(acc_sh.at[pl.ds(TAIL0, TAIL)],
                        out_hbm.at[c, pl.ds(TAIL0, TAIL)])


def _make_deg_kernel():
    @functools.partial(
        pl.kernel,
        mesh=_sc_mesh(),
        out_type=jax.ShapeDtypeStruct((NC, N, D), _F32),
        scratch_types=[
            pltpu.VMEM((CH, K), _I32),
            pltpu.VMEM((K, D), _F32),
            pltpu.VMEM_SHARED((NACC, D), _F32),
        ],
    )
    def deg_kernel(pk_hbm, ones_hbm, zeros_hbm, degp_hbm, dst_v, ones_v, acc_sh):
        c = lax.axis_index("c")
        s = lax.axis_index("s")
        w = s * NC + c
        _zero_stripe(zeros_hbm, acc_sh, s)
        pltpu.sync_copy(pk_hbm.at[w], dst_v)
        pltpu.sync_copy(ones_hbm, ones_v)

        def unpack(j, carry):
            for k in range(K // 16):
                sl = pl.ds(k * 16, 16)
                dst_v[j, sl] = dst_v[j, sl] >> 16
            return carry

        lax.fori_loop(0, CH, unpack, 0)
        plsc.subcore_barrier()

        def body(j, carry):
            pltpu.sync_copy(ones_v, acc_sh.at[dst_v.at[j]], add=True)
            return carry

        lax.fori_loop(0, CH, body, 0)
        plsc.subcore_barrier()
        _write_stripe(acc_sh, degp_hbm, c, s)

    return deg_kernel


def _make_agg_kernel():
    @functools.partial(
        pl.kernel,
        mesh=_sc_mesh(),
        out_type=jax.ShapeDtypeStruct((NC, N, D), _F32),
        scratch_types=[
            pltpu.VMEM((CH, K), _I32),
            pltpu.VMEM((2, K), _I32),
            pltpu.VMEM((2, K), _I32),
            pltpu.VMEM((2, K, D), _F32),
            pltpu.VMEM_SHARED((NACC, D), _F32),
            pltpu.SemaphoreType.DMA,
        ],
    )
    def agg_kernel(hs_hbm, pk_hbm, zeros_hbm, part_hbm, pk_v, src_c, dst_c,
                   rows_v, acc_sh, sem):
        c = lax.axis_index("c")
        s = lax.axis_index("s")
        w = s * NC + c
        _zero_stripe(zeros_hbm, acc_sh, s)
        pltpu.sync_copy(pk_hbm.at[w], pk_v)

        def unpack(j, b):
            for k in range(K // 16):
                sl = pl.ds(k * 16, 16)
                v = pk_v[j, sl]
                src_c[b, sl] = v & 0xFFFF
                dst_c[b, sl] = v >> 16

        plsc.subcore_barrier()

        unpack(0, 0)
        pltpu.make_async_copy(hs_hbm.at[src_c.at[0]], rows_v.at[0], sem).start()

        def pair(jo, carry):
            j0 = 2 * jo
            unpack(j0 + 1, 1)
            pltpu.make_async_copy(hs_hbm.at[src_c.at[0]], rows_v.at[0], sem).wait()
            pltpu.make_async_copy(hs_hbm.at[src_c.at[1]], rows_v.at[1], sem).start()
            pltpu.sync_copy(rows_v.at[0], acc_sh.at[dst_c.at[0]], add=True)
            pltpu.make_async_copy(hs_hbm.at[src_c.at[1]], rows_v.at[1], sem).wait()

            @pl.when(j0 + 2 < CH)
            def _():
                unpack(j0 + 2, 0)
                pltpu.make_async_copy(hs_hbm.at[src_c.at[0]], rows_v.at[0], sem).start()

            pltpu.sync_copy(rows_v.at[1], acc_sh.at[dst_c.at[1]], add=True)
            return carry

        lax.fori_loop(0, CH // 2, pair, 0)
        if CH % 2:
            pltpu.make_async_copy(hs_hbm.at[src_c.at[0]], rows_v.at[0], sem).wait()
            pltpu.sync_copy(rows_v.at[0], acc_sh.at[dst_c.at[0]], add=True)
        plsc.subcore_barrier()
        _write_stripe(acc_sh, part_hbm, c, s)

    return agg_kernel


_deg_kernel = _make_deg_kernel()
_agg_kernel = _make_agg_kernel()

BN = 1000


def _tc1_body(x_ref, lin_ref, degp_ref, hs_ref, dinv_ref):
    deg = 1.0 + degp_ref[0] + degp_ref[1]
    dinv = lax.rsqrt(deg)
    h = jnp.dot(x_ref[...], lin_ref[...].T, preferred_element_type=_F32)
    hs_ref[...] = h * dinv
    dinv_ref[...] = dinv[:, 0:16]


def _tc2_body(x_ref, hs_ref, part_ref, dinv_ref, W_ref, b_ref, lin2_ref,
              x1_ref, hs2_ref):
    xx = x_ref[...]
    dinv = dinv_ref[:, 0:1]
    agg = part_ref[0] + part_ref[1] + hs_ref[...]
    W = W_ref[...]
    z = (jnp.dot(xx, W.T, preferred_element_type=_F32)
         - jnp.dot(xx, W, preferred_element_type=_F32)
         - GAMMA * xx + dinv * agg + b_ref[...])
    x1 = jax.nn.relu(xx + EPS * jnp.tanh(z))
    x1_ref[...] = x1
    hs2_ref[...] = jnp.dot(x1, lin2_ref[...].T,
                           preferred_element_type=_F32) * dinv


def _tc3_body(x_ref, hs_ref, part_ref, dinv_ref, W_ref, b_ref, ow_ref, ob_ref,
              out_ref):
    xx = x_ref[...]
    dinv = dinv_ref[:, 0:1]
    agg = part_ref[0] + part_ref[1] + hs_ref[...]
    W = W_ref[...]
    z = (jnp.dot(xx, W.T, preferred_element_type=_F32)
         - jnp.dot(xx, W, preferred_element_type=_F32)
         - GAMMA * xx + dinv * agg + b_ref[...])
    x2 = jax.nn.relu(xx + EPS * jnp.tanh(z))
    out_ref[...] = jnp.dot(x2, ow_ref[...], preferred_element_type=_F32) + ob_ref[0, 0]


def _row_spec(width):
    return pl.BlockSpec((BN, width), lambda i: (i, 0))


def _full_spec(shape):
    nd = len(shape)
    return pl.BlockSpec(shape, lambda i: (0,) * nd)


def _part_spec(width):
    return pl.BlockSpec((NC, BN, width), lambda i: (0, i, 0))


def kernel(x, edge_index, W1, b1, lin1, W2, b2, lin2, out_w, out_b):
    src = edge_index[0].astype(_I32)
    dst = edge_index[1].astype(_I32)
    pk = (src | (dst << 16)).reshape(NW, CH, K)
    onesD = jnp.ones((K, D), _F32)
    zerosD = jnp.zeros((ZR, D), _F32)
    b1r = b1.reshape(1, D)
    b2r = b2.reshape(1, D)
    obr = out_b.reshape(1, 1)
    grid = (N // BN,)

    degp = _deg_kernel(pk, onesD, zerosD)

    hs1, dinv16 = pl.pallas_call(
        _tc1_body,
        grid=grid,
        in_specs=[_row_spec(D), _full_spec((D, D)), _part_spec(D)],
        out_specs=[_row_spec(D), _row_spec(16)],
        out_shape=[jax.ShapeDtypeStruct((N, D), _F32),
                   jax.ShapeDtypeStruct((N, 16), _F32)],
    )(x, lin1, degp)

    part1 = _agg_kernel(hs1, pk, zerosD)

    x1, hs2 = pl.pallas_call(
        _tc2_body,
        grid=grid,
        in_specs=[_row_spec(D), _row_spec(D), _part_spec(D), _row_spec(16),
                  _full_spec((D, D)), _full_spec((1, D)), _full_spec((D, D))],
        out_specs=[_row_spec(D), _row_spec(D)],
        out_shape=[jax.ShapeDtypeStruct((N, D), _F32),
                   jax.ShapeDtypeStruct((N, D), _F32)],
    )(x, hs1, part1, dinv16, W1, b1r, lin2)

    part2 = _agg_kernel(hs2, pk, zerosD)

    out2d = pl.pallas_call(
        _tc3_body,
        grid=grid,
        in_specs=[_row_spec(D), _row_spec(D), _part_spec(D), _row_spec(16),
                  _full_spec((D, D)), _full_spec((1, D)), _full_spec((D, 1)),
                  pl.BlockSpec(memory_space=pltpu.SMEM)],
        out_specs=pl.BlockSpec((BN, 1), lambda i: (i, 0)),
        out_shape=jax.ShapeDtypeStruct((N, 1), _F32),
    )(x1, hs2, part2, dinv16, W2, b2r, out_w.reshape(D, 1), obr)

    return out2d[:, 0]


# speedup vs baseline: 1.3007x; 1.0066x over previous
import functools

import jax
import jax.numpy as jnp
from jax import lax
from jax.experimental import pallas as pl
from jax.experimental.pallas import tpu as pltpu
from jax.experimental.pallas import tpu_sc as plsc

N = 10000
E = 320000
D = 128
EPS = 0.1
GAMMA = 0.1

NC = 2
NS = 16
NW = NC * NS
EPT = E // NW
K = 80
CH = EPT // K
NACC = N
STRIPE = 624
TAIL = N - NS * STRIPE
TAIL0 = NS * STRIPE
ZR = 208
NH = 8

_F32 = jnp.float32
_I32 = jnp.int32


def _sc_mesh():
    return plsc.VectorSubcoreMesh(core_axis_name="c", subcore_axis_name="s")


def _zero_stripe(zeros_hbm, acc_sh, s):
    row0 = s * STRIPE
    for t in range(STRIPE // ZR):
        pltpu.sync_copy(zeros_hbm, acc_sh.at[pl.ds(row0 + t * ZR, ZR)])

    @pl.when(s == 0)
    def _():
        pltpu.sync_copy(zeros_hbm.at[pl.ds(0, TAIL)], acc_sh.at[pl.ds(TAIL0, TAIL)])


def _write_stripe(acc_sh, out_hbm, c, s):
    row0 = s * STRIPE
    pltpu.sync_copy(acc_sh.at[pl.ds(row0, STRIPE)],
                    out_hbm.at[c, pl.ds(row0, STRIPE)])

    @pl.when(s == 0)
    def _():
        pltpu.sync_copy(acc_sh.at[pl.ds(TAIL0, TAIL)],
                        out_hbm.at[c, pl.ds(TAIL0, TAIL)])


def _make_deg_kernel():
    @functools.partial(
        pl.kernel,
        mesh=_sc_mesh(),
        out_type=jax.ShapeDtypeStruct((NC, N, D), _F32),
        scratch_types=[
            pltpu.VMEM((CH, K), _I32),
            pltpu.VMEM((K, D), _F32),
            pltpu.VMEM_SHARED((NACC, D), _F32),
        ],
    )
    def deg_kernel(pk_hbm, ones_hbm, zeros_hbm, degp_hbm, dst_v, ones_v, acc_sh):
        c = lax.axis_index("c")
        s = lax.axis_index("s")
        w = s * NC + c
        _zero_stripe(zeros_hbm, acc_sh, s)
        pltpu.sync_copy(pk_hbm.at[w], dst_v)
        pltpu.sync_copy(ones_hbm, ones_v)

        def unpack(j, carry):
            for k in range(K // 16):
                sl = pl.ds(k * 16, 16)
                dst_v[j, sl] = dst_v[j, sl] >> 16
            return carry

        lax.fori_loop(0, CH, unpack, 0)
        plsc.subcore_barrier()

        def body(j, carry):
            pltpu.sync_copy(ones_v, acc_sh.at[dst_v.at[j]], add=True)
            return carry

        lax.fori_loop(0, CH, body, 0)
        plsc.subcore_barrier()
        _write_stripe(acc_sh, degp_hbm, c, s)

    return deg_kernel


def _make_agg_kernel():
    @functools.partial(
        pl.kernel,
        mesh=_sc_mesh(),
        out_type=jax.ShapeDtypeStruct((NC, N, D), _F32),
        scratch_types=[
            pltpu.VMEM((CH, K), _I32),
            pltpu.VMEM((2, K), _I32),
            pltpu.VMEM((2, K), _I32),
            pltpu.VMEM((2, K, D), _F32),
            pltpu.VMEM_SHARED((NACC, D), _F32),
            pltpu.SemaphoreType.DMA,
        ],
    )
    def agg_kernel(hs_hbm, pk_hbm, zeros_hbm, part_hbm, pk_v, src_c, dst_c,
                   rows_v, acc_sh, sem):
        c = lax.axis_index("c")
        s = lax.axis_index("s")
        w = s * NC + c
        _zero_stripe(zeros_hbm, acc_sh, s)
        pltpu.sync_copy(pk_hbm.at[w], pk_v)

        def unpack(j, b):
            for k in range(K // 16):
                sl = pl.ds(k * 16, 16)
                v = pk_v[j, sl]
                src_c[b, sl] = v & 0xFFFF
                dst_c[b, sl] = v >> 16

        plsc.subcore_barrier()

        unpack(0, 0)
        pltpu.make_async_copy(hs_hbm.at[src_c.at[0]], rows_v.at[0], sem).start()

        def pair(jo, carry):
            j0 = 2 * jo
            unpack(j0 + 1, 1)
            pltpu.make_async_copy(hs_hbm.at[src_c.at[0]], rows_v.at[0], sem).wait()
            pltpu.make_async_copy(hs_hbm.at[src_c.at[1]], rows_v.at[1], sem).start()
            pltpu.sync_copy(rows_v.at[0], acc_sh.at[dst_c.at[0]], add=True)
            pltpu.make_async_copy(hs_hbm.at[src_c.at[1]], rows_v.at[1], sem).wait()

            @pl.when(j0 + 2 < CH)
            def _():
                unpack(j0 + 2, 0)
                pltpu.make_async_copy(hs_hbm.at[src_c.at[0]], rows_v.at[0], sem).start()

            pltpu.sync_copy(rows_v.at[1], acc_sh.at[dst_c.at[1]], add=True)
            return carry

        lax.fori_loop(0, CH // 2, pair, 0)
        if CH % 2:
            pltpu.make_async_copy(hs_hbm.at[src_c.at[0]], rows_v.at[0], sem).wait()
            pltpu.sync_copy(rows_v.at[0], acc_sh.at[dst_c.at[0]], add=True)
        plsc.subcore_barrier()
        _write_stripe(acc_sh, part_hbm, c, s)

    return agg_kernel


_deg_kernel = _make_deg_kernel()
_agg_kernel = _make_agg_kernel()

BN = 10000


def _tc1_body(x_ref, lin_ref, degp_ref, hs_ref, dinv_ref):
    deg = 1.0 + degp_ref[0] + degp_ref[1]
    dinv = lax.rsqrt(deg)
    h = jnp.dot(x_ref[...], lin_ref[...].T, preferred_element_type=_F32)
    hs_ref[...] = h * dinv
    dinv_ref[...] = dinv[:, 0:16]


def _tc2_body(x_ref, hs_ref, part_ref, dinv_ref, W_ref, b_ref, lin2_ref,
              x1_ref, hs2_ref):
    xx = x_ref[...]
    dinv = dinv_ref[:, 0:1]
    agg = part_ref[0] + part_ref[1] + hs_ref[...]
    W = W_ref[...]
    z = (jnp.dot(xx, W.T, preferred_element_type=_F32)
         - jnp.dot(xx, W, preferred_element_type=_F32)
         - GAMMA * xx + dinv * agg + b_ref[...])
    x1 = jax.nn.relu(xx + EPS * jnp.tanh(z))
    x1_ref[...] = x1
    hs2_ref[...] = jnp.dot(x1, lin2_ref[...].T,
                           preferred_element_type=_F32) * dinv


def _tc3_body(x_ref, hs_ref, part_ref, dinv_ref, W_ref, b_ref, ow_ref, ob_ref,
              out_ref):
    xx = x_ref[...]
    dinv = dinv_ref[:, 0:1]
    agg = part_ref[0] + part_ref[1] + hs_ref[...]
    W = W_ref[...]
    z = (jnp.dot(xx, W.T, preferred_element_type=_F32)
         - jnp.dot(xx, W, preferred_element_type=_F32)
         - GAMMA * xx + dinv * agg + b_ref[...])
    x2 = jax.nn.relu(xx + EPS * jnp.tanh(z))
    out_ref[...] = jnp.dot(x2, ow_ref[...], preferred_element_type=_F32) + ob_ref[0, 0]


def _row_spec(width):
    return pl.BlockSpec((BN, width), lambda i: (i, 0))


def _full_spec(shape):
    nd = len(shape)
    return pl.BlockSpec(shape, lambda i: (0,) * nd)


def _part_spec(width):
    return pl.BlockSpec((NC, BN, width), lambda i: (0, i, 0))


def kernel(x, edge_index, W1, b1, lin1, W2, b2, lin2, out_w, out_b):
    src = edge_index[0].astype(_I32)
    dst = edge_index[1].astype(_I32)
    pk = (src | (dst << 16)).reshape(NW, CH, K)
    onesD = jnp.ones((K, D), _F32)
    zerosD = jnp.zeros((ZR, D), _F32)
    b1r = b1.reshape(1, D)
    b2r = b2.reshape(1, D)
    obr = out_b.reshape(1, 1)
    grid = (N // BN,)

    degp = _deg_kernel(pk, onesD, zerosD)

    hs1, dinv16 = pl.pallas_call(
        _tc1_body,
        grid=grid,
        in_specs=[_row_spec(D), _full_spec((D, D)), _part_spec(D)],
        out_specs=[_row_spec(D), _row_spec(16)],
        out_shape=[jax.ShapeDtypeStruct((N, D), _F32),
                   jax.ShapeDtypeStruct((N, 16), _F32)],
    )(x, lin1, degp)

    part1 = _agg_kernel(hs1, pk, zerosD)

    x1, hs2 = pl.pallas_call(
        _tc2_body,
        grid=grid,
        in_specs=[_row_spec(D), _row_spec(D), _part_spec(D), _row_spec(16),
                  _full_spec((D, D)), _full_spec((1, D)), _full_spec((D, D))],
        out_specs=[_row_spec(D), _row_spec(D)],
        out_shape=[jax.ShapeDtypeStruct((N, D), _F32),
                   jax.ShapeDtypeStruct((N, D), _F32)],
    )(x, hs1, part1, dinv16, W1, b1r, lin2)

    part2 = _agg_kernel(hs2, pk, zerosD)

    out2d = pl.pallas_call(
        _tc3_body,
        grid=grid,
        in_specs=[_row_spec(D), _row_spec(D), _part_spec(D), _row_spec(16),
                  _full_spec((D, D)), _full_spec((1, D)), _full_spec((D, 1)),
                  pl.BlockSpec(memory_space=pltpu.SMEM)],
        out_specs=pl.BlockSpec((BN, 1), lambda i: (i, 0)),
        out_shape=jax.ShapeDtypeStruct((N, 1), _F32),
    )(x1, hs2, part2, dinv16, W2, b2r, out_w.reshape(D, 1), obr)

    return out2d[:, 0]


# speedup vs baseline: 1.3090x; 1.0063x over previous
import functools

import jax
import jax.numpy as jnp
from jax import lax
from jax.experimental import pallas as pl
from jax.experimental.pallas import tpu as pltpu
from jax.experimental.pallas import tpu_sc as plsc

N = 10000
E = 320000
D = 128
EPS = 0.1
GAMMA = 0.1

NC = 2
NS = 16
NW = NC * NS
EPT = E // NW
K = 80
CH = EPT // K
NACC = N
STRIPE = 624
TAIL = N - NS * STRIPE
TAIL0 = NS * STRIPE
ZR = 208
NH = 8

_F32 = jnp.float32
_I32 = jnp.int32


def _sc_mesh():
    return plsc.VectorSubcoreMesh(core_axis_name="c", subcore_axis_name="s")


def _zero_stripe(zeros_hbm, acc_sh, s):
    row0 = s * STRIPE
    for t in range(STRIPE // ZR):
        pltpu.sync_copy(zeros_hbm, acc_sh.at[pl.ds(row0 + t * ZR, ZR)])

    @pl.when(s == 0)
    def _():
        pltpu.sync_copy(zeros_hbm.at[pl.ds(0, TAIL)], acc_sh.at[pl.ds(TAIL0, TAIL)])


def _write_stripe(acc_sh, out_hbm, c, s):
    row0 = s * STRIPE
    pltpu.sync_copy(acc_sh.at[pl.ds(row0, STRIPE)],
                    out_hbm.at[c, pl.ds(row0, STRIPE)])

    @pl.when(s == 0)
    def _():
        pltpu.sync_copy(acc_sh.at[pl.ds(TAIL0, TAIL)],
                        out_hbm.at[c, pl.ds(TAIL0, TAIL)])


def _make_deg_kernel():
    @functools.partial(
        pl.kernel,
        mesh=_sc_mesh(),
        out_type=jax.ShapeDtypeStruct((NC, N, D), _F32),
        scratch_types=[
            pltpu.VMEM((CH, K), _I32),
            pltpu.VMEM((K, D), _F32),
            pltpu.VMEM_SHARED((NACC, D), _F32),
        ],
    )
    def deg_kernel(pk_hbm, ones_hbm, zeros_hbm, degp_hbm, dst_v, ones_v, acc_sh):
        c = lax.axis_index("c")
        s = lax.axis_index("s")
        w = s * NC + c
        _zero_stripe(zeros_hbm, acc_sh, s)
        pltpu.sync_copy(pk_hbm.at[w], dst_v)
        pltpu.sync_copy(ones_hbm, ones_v)

        def unpack(j, carry):
            for k in range(K // 16):
                sl = pl.ds(k * 16, 16)
                dst_v[j, sl] = dst_v[j, sl] >> 16
            return carry

        lax.fori_loop(0, CH, unpack, 0)
        plsc.subcore_barrier()

        def body(j, carry):
            pltpu.sync_copy(ones_v, acc_sh.at[dst_v.at[j]], add=True)
            return carry

        lax.fori_loop(0, CH, body, 0)
        plsc.subcore_barrier()
        _write_stripe(acc_sh, degp_hbm, c, s)

    return deg_kernel


def _make_agg_kernel():
    @functools.partial(
        pl.kernel,
        mesh=_sc_mesh(),
        out_type=jax.ShapeDtypeStruct((NC, N, D), _F32),
        scratch_types=[
            pltpu.VMEM((CH, K), _I32),
            pltpu.VMEM((2, K), _I32),
            pltpu.VMEM((2, K), _I32),
            pltpu.VMEM((2, K, D), _F32),
            pltpu.VMEM_SHARED((NACC, D), _F32),
            pltpu.SemaphoreType.DMA,
        ],
    )
    def agg_kernel(hs_hbm, pk_hbm, zeros_hbm, part_hbm, pk_v, src_c, dst_c,
                   rows_v, acc_sh, sem):
        c = lax.axis_index("c")
        s = lax.axis_index("s")
        w = s * NC + c
        _zero_stripe(zeros_hbm, acc_sh, s)
        pltpu.sync_copy(pk_hbm.at[w], pk_v)

        def unpack(j, b):
            for k in range(K // 16):
                sl = pl.ds(k * 16, 16)
                v = pk_v[j, sl]
                src_c[b, sl] = v & 0xFFFF
                dst_c[b, sl] = v >> 16

        plsc.subcore_barrier()

        unpack(0, 0)
        pltpu.make_async_copy(hs_hbm.at[src_c.at[0]], rows_v.at[0], sem).start()

        def pair(jo, carry):
            j0 = 2 * jo
            unpack(j0 + 1, 1)
            pltpu.make_async_copy(hs_hbm.at[src_c.at[0]], rows_v.at[0], sem).wait()
            pltpu.make_async_copy(hs_hbm.at[src_c.at[1]], rows_v.at[1], sem).start()
            pltpu.sync_copy(rows_v.at[0], acc_sh.at[dst_c.at[0]], add=True)
            pltpu.make_async_copy(hs_hbm.at[src_c.at[1]], rows_v.at[1], sem).wait()

            @pl.when(j0 + 2 < CH)
            def _():
                unpack(j0 + 2, 0)
                pltpu.make_async_copy(hs_hbm.at[src_c.at[0]], rows_v.at[0], sem).start()

            pltpu.sync_copy(rows_v.at[1], acc_sh.at[dst_c.at[1]], add=True)
            return carry

        lax.fori_loop(0, CH // 2, pair, 0)
        if CH % 2:
            pltpu.make_async_copy(hs_hbm.at[src_c.at[0]], rows_v.at[0], sem).wait()
            pltpu.sync_copy(rows_v.at[0], acc_sh.at[dst_c.at[0]], add=True)
        plsc.subcore_barrier()
        _write_stripe(acc_sh, part_hbm, c, s)

    return agg_kernel


_deg_kernel = _make_deg_kernel()
_agg_kernel = _make_agg_kernel()

BN = 2000


def _tc1_body(x_ref, lin_ref, degp_ref, hs_ref, dinv_ref):
    deg = 1.0 + degp_ref[0] + degp_ref[1]
    dinv = lax.rsqrt(deg)
    h = jnp.dot(x_ref[...], lin_ref[...].T, preferred_element_type=_F32)
    hs_ref[...] = h * dinv
    dinv_ref[...] = dinv[:, 0:16]


def _tc2_body(x_ref, hs_ref, part_ref, dinv_ref, W_ref, b_ref, lin2_ref,
              x1_ref, hs2_ref):
    xx = x_ref[...]
    dinv = dinv_ref[:, 0:1]
    agg = part_ref[0] + part_ref[1] + hs_ref[...]
    W = W_ref[...]
    z = (jnp.dot(xx, W.T, preferred_element_type=_F32)
         - jnp.dot(xx, W, preferred_element_type=_F32)
         - GAMMA * xx + dinv * agg + b_ref[...])
    x1 = jax.nn.relu(xx + EPS * jnp.tanh(z))
    x1_ref[...] = x1
    hs2_ref[...] = jnp.dot(x1, lin2_ref[...].T,
                           preferred_element_type=_F32) * dinv


def _tc3_body(x_ref, hs_ref, part_ref, dinv_ref, W_ref, b_ref, ow_ref, ob_ref,
              out_ref):
    xx = x_ref[...]
    dinv = dinv_ref[:, 0:1]
    agg = part_ref[0] + part_ref[1] + hs_ref[...]
    W = W_ref[...]
    z = (jnp.dot(xx, W.T, preferred_element_type=_F32)
         - jnp.dot(xx, W, preferred_element_type=_F32)
         - GAMMA * xx + dinv * agg + b_ref[...])
    x2 = jax.nn.relu(xx + EPS * jnp.tanh(z))
    out_ref[...] = jnp.dot(x2, ow_ref[...], preferred_element_type=_F32) + ob_ref[0, 0]


def _row_spec(width):
    return pl.BlockSpec((BN, width), lambda i: (i, 0))


def _full_spec(shape):
    nd = len(shape)
    return pl.BlockSpec(shape, lambda i: (0,) * nd)


def _part_spec(width):
    return pl.BlockSpec((NC, BN, width), lambda i: (0, i, 0))


def kernel(x, edge_index, W1, b1, lin1, W2, b2, lin2, out_w, out_b):
    src = edge_index[0].astype(_I32)
    dst = edge_index[1].astype(_I32)
    pk = (src | (dst << 16)).reshape(NW, CH, K)
    onesD = jnp.ones((K, D), _F32)
    zerosD = jnp.zeros((ZR, D), _F32)
    b1r = b1.reshape(1, D)
    b2r = b2.reshape(1, D)
    obr = out_b.reshape(1, 1)
    grid = (N // BN,)

    degp = _deg_kernel(pk, onesD, zerosD)

    hs1, dinv16 = pl.pallas_call(
        _tc1_body,
        grid=grid,
        in_specs=[_row_spec(D), _full_spec((D, D)), _part_spec(D)],
        out_specs=[_row_spec(D), _row_spec(16)],
        out_shape=[jax.ShapeDtypeStruct((N, D), _F32),
                   jax.ShapeDtypeStruct((N, 16), _F32)],
    )(x, lin1, degp)

    part1 = _agg_kernel(hs1, pk, zerosD)

    x1, hs2 = pl.pallas_call(
        _tc2_body,
        grid=grid,
        in_specs=[_row_spec(D), _row_spec(D), _part_spec(D), _row_spec(16),
                  _full_spec((D, D)), _full_spec((1, D)), _full_spec((D, D))],
        out_specs=[_row_spec(D), _row_spec(D)],
        out_shape=[jax.ShapeDtypeStruct((N, D), _F32),
                   jax.ShapeDtypeStruct((N, D), _F32)],
    )(x, hs1, part1, dinv16, W1, b1r, lin2)

    part2 = _agg_kernel(hs2, pk, zerosD)

    out2d = pl.pallas_call(
        _tc3_body,
        grid=grid,
        in_specs=[_row_spec(D), _row_spec(D), _part_spec(D), _row_spec(16),
                  _full_spec((D, D)), _full_spec((1, D)), _full_spec((D, 1)),
                  pl.BlockSpec(memory_space=pltpu.SMEM)],
        out_specs=pl.BlockSpec((BN, 1), lambda i: (i, 0)),
        out_shape=jax.ShapeDtypeStruct((N, 1), _F32),
    )(x1, hs2, part2, dinv16, W2, b2r, out_w.reshape(D, 1), obr)

    return out2d[:, 0]


# speedup vs baseline: 1.3415x; 1.0248x over previous
import functools

import jax
import jax.numpy as jnp
from jax import lax
from jax.experimental import pallas as pl
from jax.experimental.pallas import tpu as pltpu
from jax.experimental.pallas import tpu_sc as plsc

N = 10000
E = 320000
D = 128
EPS = 0.1
GAMMA = 0.1

NC = 2
NS = 16
NW = NC * NS
EPT = E // NW
K = 80
CH = EPT // K
NACC = N
STRIPE = 624
TAIL = N - NS * STRIPE
TAIL0 = NS * STRIPE
ZR = 208
NH = 8

_F32 = jnp.float32
_I32 = jnp.int32


def _sc_mesh():
    return plsc.VectorSubcoreMesh(core_axis_name="c", subcore_axis_name="s")


def _zero_stripe(zeros_hbm, acc_sh, s):
    row0 = s * STRIPE
    for t in range(STRIPE // ZR):
        pltpu.sync_copy(zeros_hbm, acc_sh.at[pl.ds(row0 + t * ZR, ZR)])

    @pl.when(s == 0)
    def _():
        pltpu.sync_copy(zeros_hbm.at[pl.ds(0, TAIL)], acc_sh.at[pl.ds(TAIL0, TAIL)])


def _write_stripe(acc_sh, out_hbm, c, s):
    row0 = s * STRIPE
    pltpu.sync_copy(acc_sh.at[pl.ds(row0, STRIPE)],
                    out_hbm.at[c, pl.ds(row0, STRIPE)])

    @pl.when(s == 0)
    def _():
        pltpu.sync_copy(acc_sh.at[pl.ds(TAIL0, TAIL)],
                        out_hbm.at[c, pl.ds(TAIL0, TAIL)])


def _make_deg_kernel():
    @functools.partial(
        pl.kernel,
        mesh=_sc_mesh(),
        out_type=jax.ShapeDtypeStruct((NC, N, D), _F32),
        scratch_types=[
            pltpu.VMEM((CH, K), _I32),
            pltpu.VMEM((K, D), _F32),
            pltpu.VMEM_SHARED((NACC, D), _F32),
        ],
    )
    def deg_kernel(pk_hbm, ones_hbm, zeros_hbm, degp_hbm, dst_v, ones_v, acc_sh):
        c = lax.axis_index("c")
        s = lax.axis_index("s")
        w = s * NC + c
        _zero_stripe(zeros_hbm, acc_sh, s)
        pltpu.sync_copy(pk_hbm.at[w], dst_v)
        pltpu.sync_copy(ones_hbm, ones_v)

        def unpack(j, carry):
            for k in range(K // 16):
                sl = pl.ds(k * 16, 16)
                dst_v[j, sl] = dst_v[j, sl] >> 16
            return carry

        lax.fori_loop(0, CH, unpack, 0)
        plsc.subcore_barrier()

        def body(j, carry):
            pltpu.sync_copy(ones_v, acc_sh.at[dst_v.at[j]], add=True)
            return carry

        lax.fori_loop(0, CH, body, 0)
        plsc.subcore_barrier()
        _write_stripe(acc_sh, degp_hbm, c, s)

    return deg_kernel


def _make_agg_kernel():
    @functools.partial(
        pl.kernel,
        mesh=_sc_mesh(),
        out_type=jax.ShapeDtypeStruct((NC, N, D), _F32),
        scratch_types=[
            pltpu.VMEM((CH, K), _I32),
            pltpu.VMEM((2, K), _I32),
            pltpu.VMEM((2, K), _I32),
            pltpu.VMEM((2, K, D), _F32),
            pltpu.VMEM_SHARED((NACC, D), _F32),
            pltpu.SemaphoreType.DMA,
        ],
    )
    def agg_kernel(hs_hbm, pk_hbm, zeros_hbm, part_hbm, pk_v, src_c, dst_c,
                   rows_v, acc_sh, sem):
        c = lax.axis_index("c")
        s = lax.axis_index("s")
        w = s * NC + c
        row0 = s * STRIPE

        @pl.when(c == 0)
        def _():
            for t in range(STRIPE // ZR):
                sl = pl.ds(row0 + t * ZR, ZR)
                pltpu.sync_copy(hs_hbm.at[sl], acc_sh.at[sl])

            @pl.when(s == 0)
            def _():
                sl = pl.ds(TAIL0, TAIL)
                pltpu.sync_copy(hs_hbm.at[sl], acc_sh.at[sl])

        @pl.when(c == 1)
        def _():
            _zero_stripe(zeros_hbm, acc_sh, s)

        pltpu.sync_copy(pk_hbm.at[w], pk_v)

        def unpack(j, b):
            for k in range(K // 16):
                sl = pl.ds(k * 16, 16)
                v = pk_v[j, sl]
                src_c[b, sl] = v & 0xFFFF
                dst_c[b, sl] = v >> 16

        plsc.subcore_barrier()

        unpack(0, 0)
        pltpu.make_async_copy(hs_hbm.at[src_c.at[0]], rows_v.at[0], sem).start()

        def pair(jo, carry):
            j0 = 2 * jo
            unpack(j0 + 1, 1)
            pltpu.make_async_copy(hs_hbm.at[src_c.at[0]], rows_v.at[0], sem).wait()
            pltpu.make_async_copy(hs_hbm.at[src_c.at[1]], rows_v.at[1], sem).start()
            pltpu.sync_copy(rows_v.at[0], acc_sh.at[dst_c.at[0]], add=True)
            pltpu.make_async_copy(hs_hbm.at[src_c.at[1]], rows_v.at[1], sem).wait()

            @pl.when(j0 + 2 < CH)
            def _():
                unpack(j0 + 2, 0)
                pltpu.make_async_copy(hs_hbm.at[src_c.at[0]], rows_v.at[0], sem).start()

            pltpu.sync_copy(rows_v.at[1], acc_sh.at[dst_c.at[1]], add=True)
            return carry

        lax.fori_loop(0, CH // 2, pair, 0)
        if CH % 2:
            pltpu.make_async_copy(hs_hbm.at[src_c.at[0]], rows_v.at[0], sem).wait()
            pltpu.sync_copy(rows_v.at[0], acc_sh.at[dst_c.at[0]], add=True)
        plsc.subcore_barrier()
        _write_stripe(acc_sh, part_hbm, c, s)

    return agg_kernel


_deg_kernel = _make_deg_kernel()
_agg_kernel = _make_agg_kernel()

BN = 2000


def _tc1_body(x_ref, lin_ref, degp_ref, hs_ref, dinv_ref):
    deg = 1.0 + degp_ref[0] + degp_ref[1]
    dinv = lax.rsqrt(deg)
    h = jnp.dot(x_ref[...], lin_ref[...].T, preferred_element_type=_F32)
    hs_ref[...] = h * dinv
    dinv_ref[...] = dinv[:, 0:16]


def _tc2_body(x_ref, part_ref, dinv_ref, W_ref, b_ref, lin2_ref,
              x1_ref, hs2_ref):
    xx = x_ref[...]
    dinv = dinv_ref[:, 0:1]
    agg = part_ref[0] + part_ref[1]
    W = W_ref[...]
    z = (jnp.dot(xx, W.T, preferred_element_type=_F32)
         - jnp.dot(xx, W, preferred_element_type=_F32)
         - GAMMA * xx + dinv * agg + b_ref[...])
    x1 = jax.nn.relu(xx + EPS * jnp.tanh(z))
    x1_ref[...] = x1
    hs2_ref[...] = jnp.dot(x1, lin2_ref[...].T,
                           preferred_element_type=_F32) * dinv


def _tc3_body(x_ref, part_ref, dinv_ref, W_ref, b_ref, ow_ref, ob_ref,
              out_ref):
    xx = x_ref[...]
    dinv = dinv_ref[:, 0:1]
    agg = part_ref[0] + part_ref[1]
    W = W_ref[...]
    z = (jnp.dot(xx, W.T, preferred_element_type=_F32)
         - jnp.dot(xx, W, preferred_element_type=_F32)
         - GAMMA * xx + dinv * agg + b_ref[...])
    x2 = jax.nn.relu(xx + EPS * jnp.tanh(z))
    out_ref[...] = jnp.dot(x2, ow_ref[...], preferred_element_type=_F32) + ob_ref[0, 0]


def _row_spec(width):
    return pl.BlockSpec((BN, width), lambda i: (i, 0))


def _full_spec(shape):
    nd = len(shape)
    return pl.BlockSpec(shape, lambda i: (0,) * nd)


def _part_spec(width):
    return pl.BlockSpec((NC, BN, width), lambda i: (0, i, 0))


def kernel(x, edge_index, W1, b1, lin1, W2, b2, lin2, out_w, out_b):
    src = edge_index[0].astype(_I32)
    dst = edge_index[1].astype(_I32)
    pk = (src | (dst << 16)).reshape(NW, CH, K)
    onesD = jnp.ones((K, D), _F32)
    zerosD = jnp.zeros((ZR, D), _F32)
    b1r = b1.reshape(1, D)
    b2r = b2.reshape(1, D)
    obr = out_b.reshape(1, 1)
    grid = (N // BN,)

    degp = _deg_kernel(pk, onesD, zerosD)

    hs1, dinv16 = pl.pallas_call(
        _tc1_body,
        grid=grid,
        in_specs=[_row_spec(D), _full_spec((D, D)), _part_spec(D)],
        out_specs=[_row_spec(D), _row_spec(16)],
        out_shape=[jax.ShapeDtypeStruct((N, D), _F32),
                   jax.ShapeDtypeStruct((N, 16), _F32)],
    )(x, lin1, degp)

    part1 = _agg_kernel(hs1, pk, zerosD)

    x1, hs2 = pl.pallas_call(
        _tc2_body,
        grid=grid,
        in_specs=[_row_spec(D), _part_spec(D), _row_spec(16),
                  _full_spec((D, D)), _full_spec((1, D)), _full_spec((D, D))],
        out_specs=[_row_spec(D), _row_spec(D)],
        out_shape=[jax.ShapeDtypeStruct((N, D), _F32),
                   jax.ShapeDtypeStruct((N, D), _F32)],
    )(x, part1, dinv16, W1, b1r, lin2)

    part2 = _agg_kernel(hs2, pk, zerosD)

    out2d = pl.pallas_call(
        _tc3_body,
        grid=grid,
        in_specs=[_row_spec(D), _part_spec(D), _row_spec(16),
                  _full_spec((D, D)), _full_spec((1, D)), _full_spec((D, 1)),
                  pl.BlockSpec(memory_space=pltpu.SMEM)],
        out_specs=pl.BlockSpec((BN, 1), lambda i: (i, 0)),
        out_shape=jax.ShapeDtypeStruct((N, 1), _F32),
    )(x1, part2, dinv16, W2, b2r, out_w.reshape(D, 1), obr)

    return out2d[:, 0]
